# baseline (speedup 1.0000x reference)
import jax
import jax.numpy as jnp
from jax import lax
from jax.experimental import pallas as pl
from jax.experimental.pallas import tpu as pltpu

EMB = 1024
NUM_EXPERTS = 8
TOKEN_TILE = 512


def _moe_kernel(x_ref, wg_ref, bg_ref, we_ref, be_ref, out_ref):
    xb = x_ref[...]
    s = jnp.dot(xb, wg_ref[...], preferred_element_type=jnp.float32)
    s = s + bg_ref[...]
    t = s.shape[0]
    ids = lax.broadcasted_iota(jnp.int32, (t, NUM_EXPERTS), 1)
    m1 = jnp.max(s, axis=1, keepdims=True)
    first_idx = jnp.min(jnp.where(s == m1, ids, NUM_EXPERTS), axis=1, keepdims=True)
    first_sel = ids == first_idx
    s2 = jnp.where(first_sel, -jnp.inf, s)
    m2 = jnp.max(s2, axis=1, keepdims=True)
    second_idx = jnp.min(jnp.where(s2 == m2, ids, NUM_EXPERTS), axis=1, keepdims=True)
    second_sel = ids == second_idx
    z = jnp.exp(m2 - m1)
    w1 = 1.0 / (1.0 + z)
    w2 = z * w1
    w = jnp.where(first_sel, w1, 0.0) + jnp.where(second_sel, w2, 0.0)

    acc = jnp.dot(w, be_ref[...], preferred_element_type=jnp.float32)
    xb16 = xb.astype(jnp.bfloat16)
    for e in range(NUM_EXPERTS):
        y = jnp.dot(xb16, we_ref[e], preferred_element_type=jnp.float32)
        acc = acc + w[:, e : e + 1] * y
    out_ref[...] = acc


def kernel(x, Wg, bg, We, be):
    orig_ndim = x.ndim
    if orig_ndim == 2:
        x = x[:, None, :]
    b, s, d = x.shape
    n = b * s
    xf = x.reshape(n, d)
    we16 = We.astype(jnp.bfloat16)
    tiles = pl.cdiv(n, TOKEN_TILE)

    out = pl.pallas_call(
        _moe_kernel,
        grid=(tiles,),
        in_specs=[
            pl.BlockSpec((TOKEN_TILE, d), lambda t: (t, 0)),
            pl.BlockSpec((d, NUM_EXPERTS), lambda t: (0, 0)),
            pl.BlockSpec((NUM_EXPERTS,), lambda t: (0,)),
            pl.BlockSpec((NUM_EXPERTS, d, d), lambda t: (0, 0, 0)),
            pl.BlockSpec((NUM_EXPERTS, d), lambda t: (0, 0)),
        ],
        out_specs=pl.BlockSpec((TOKEN_TILE, d), lambda t: (t, 0)),
        out_shape=jax.ShapeDtypeStruct((n, d), jnp.float32),
        compiler_params=pltpu.CompilerParams(
            dimension_semantics=("arbitrary",),
        ),
    )(xf, Wg, bg, we16, be)

    out = out.reshape(b, s, d)
    if orig_ndim == 2:
        out = out[:, 0, :]
    return out
